# baseline (speedup 1.0000x reference)
import jax
import jax.numpy as jnp
from jax.experimental import pallas as pl
from jax.experimental.pallas import tpu as pltpu
from jax.experimental.pallas import tpu_sc as plsc

_B = 1000
_GATHER_W = 256
_SCALE = 30.0
_EPS = 1e-5


def _gelu(x):
    return 0.5 * x * (1.0 + jax.lax.erf(x * 0.7071067811865476))


def _ln(x, w, b):
    mu = jnp.mean(x, axis=-1, keepdims=True)
    var = jnp.mean((x - mu) ** 2, axis=-1, keepdims=True)
    return (x - mu) / jnp.sqrt(var + _EPS) * w + b


def _sc_gather(table, idx_flat):
    m, c = idx_flat.shape[0], table.shape[1]
    mesh = plsc.VectorSubcoreMesh(core_axis_name="c", subcore_axis_name="s")
    idx2 = idx_flat.reshape(1, m)

    @pl.kernel(out_type=jax.ShapeDtypeStruct((m, c), table.dtype), mesh=mesh)
    def _gather(x_hbm, i_hbm, o_hbm):
        def body(i_vmem, o_vmem):
            pltpu.sync_copy(x_hbm.at[i_vmem.at[0]], o_vmem)

        pltpu.emit_pipeline(
            body,
            grid=(m // _GATHER_W,),
            in_specs=[pl.BlockSpec((1, _GATHER_W), index_map=lambda i: (0, i))],
            out_specs=[pl.BlockSpec((_GATHER_W, c), index_map=lambda i: (i, 0))],
            core_axis_name=("c", "s"),
            dimension_semantics=(pltpu.PARALLEL,),
        )(i_hbm, o_hbm)

    return _gather(table, idx2)


def _decoder_body(h_ref, e_ref, g_ref, m_ref,
                  whT, wegT, w1T, w2T, b0, b1, b2,
                  d0T, db0, d1T, db1, n1w, n1b, n2w, n2b,
                  o_ref):
    k, b, c = e_ref.shape
    kb = k * b
    f32 = jnp.float32
    bf = jnp.bfloat16
    h = h_ref[...]
    e2 = e_ref[...].reshape(kb, c)
    g2 = g_ref[...].reshape(kb, c).astype(bf)
    eg = jnp.concatenate([e2, g2], axis=1)
    msk = m_ref[...]
    n_layers = whT.shape[0]
    for l in range(n_layers):
        hw = jnp.dot(h.astype(bf), whT[l], preferred_element_type=f32).astype(bf)
        epre = jnp.dot(eg, wegT[l], preferred_element_type=f32).astype(bf)
        x = epre.reshape(k, b, c) + hw[None] + b0[l][None]
        m1 = _gelu(x).reshape(kb, c)
        m2 = _gelu((jnp.dot(m1, w1T[l], preferred_element_type=f32)).astype(bf) + b1[l])
        m3 = jnp.dot(m2, w2T[l], preferred_element_type=f32).astype(bf)
        agg = (jnp.sum(m3.reshape(k, b, c), axis=0).astype(f32)
               + k * b2[l]) * (1.0 / _SCALE)
        h1 = _ln(h + agg, n1w[l], n1b[l])
        d = _gelu((jnp.dot(h1.astype(bf), d0T[l], preferred_element_type=f32)).astype(bf) + db0[l])
        d = jnp.dot(d, d1T[l], preferred_element_type=f32) + db1[l]
        h = _ln(h1 + d, n2w[l], n2b[l]) * msk
    o_ref[...] = h


def _run_decoder(h0, eds, gns, msk, w, base, chunk):
    n, c = h0.shape
    k = eds.shape[0]
    nb = chunk // _B

    def full3(a):
        return pl.BlockSpec(a.shape, lambda i: (0,) * a.ndim)

    in_specs = [
        pl.BlockSpec((_B, c), lambda i: (base + i, 0)),
        pl.BlockSpec((k, _B, eds.shape[2]), lambda i: (0, base + i, 0)),
        pl.BlockSpec((k, _B, gns.shape[2]), lambda i: (0, i, 0)),
        pl.BlockSpec((_B, 1), lambda i: (base + i, 0)),
    ] + [full3(a) for a in w]

    return pl.pallas_call(
        _decoder_body,
        grid=(nb,),
        in_specs=in_specs,
        out_specs=pl.BlockSpec((_B, c), lambda i: (i, 0)),
        out_shape=jax.ShapeDtypeStruct((chunk, c), h0.dtype),
        compiler_params=pltpu.CompilerParams(
            dimension_semantics=("parallel",),
        ),
    )(h0, eds, gns, msk, *w)


_N_CHUNKS = 5


def kernel(node_features, edge_features, neighbor_indices, mask, params):
    n, c = node_features.shape
    _, k, e = edge_features.shape

    eds = edge_features.transpose(1, 0, 2).astype(jnp.bfloat16)
    msk = mask.astype(node_features.dtype).reshape(n, 1)

    st = jnp.stack
    stb = lambda xs: jnp.stack(xs).astype(jnp.bfloat16)
    w = [
        stb([p['mW0'][:, :c].T for p in params]),
        stb([jnp.concatenate([p['mW0'][:, c:c + e],
                              p['mW0'][:, c + e + c:]], axis=1).T
             for p in params]),
        stb([p['mW1'].T for p in params]),
        stb([p['mW2'].T for p in params]),
        stb([p['mb0'][None] for p in params]),
        stb([p['mb1'][None] for p in params]),
        st([p['mb2'][None] for p in params]),
        stb([p['dW0'].T for p in params]),
        stb([p['db0'][None] for p in params]),
        stb([p['dW1'].T for p in params]),
        st([p['db1'][None] for p in params]),
        st([p['n1w'][None] for p in params]),
        st([p['n1b'][None] for p in params]),
        st([p['n2w'][None] for p in params]),
        st([p['n2b'][None] for p in params]),
    ]

    idx = neighbor_indices.T.reshape(-1).astype(jnp.int32)
    gns = _sc_gather(node_features, idx).reshape(k, n, c)
    return _run_decoder(node_features, eds, gns, msk, w, 0, n)


# speedup vs baseline: 1.0059x; 1.0059x over previous
import jax
import jax.numpy as jnp
from jax.experimental import pallas as pl
from jax.experimental.pallas import tpu as pltpu
from jax.experimental.pallas import tpu_sc as plsc

_B = 1000
_GATHER_W = 128
_SCALE = 30.0
_EPS = 1e-5


def _gelu(x):
    return 0.5 * x * (1.0 + jax.lax.erf(x * 0.7071067811865476))


def _ln(x, w, b):
    mu = jnp.mean(x, axis=-1, keepdims=True)
    var = jnp.mean((x - mu) ** 2, axis=-1, keepdims=True)
    return (x - mu) / jnp.sqrt(var + _EPS) * w + b


def _sc_gather(table, idx_flat):
    m, c = idx_flat.shape[0], table.shape[1]
    mesh = plsc.VectorSubcoreMesh(core_axis_name="c", subcore_axis_name="s")
    idx2 = idx_flat.reshape(1, m)

    @pl.kernel(out_type=jax.ShapeDtypeStruct((m, c), table.dtype), mesh=mesh)
    def _gather(x_hbm, i_hbm, o_hbm):
        def body(i_vmem, o_vmem):
            pltpu.sync_copy(x_hbm.at[i_vmem.at[0]], o_vmem)

        pltpu.emit_pipeline(
            body,
            grid=(m // _GATHER_W,),
            in_specs=[pl.BlockSpec((1, _GATHER_W), index_map=lambda i: (0, i))],
            out_specs=[pl.BlockSpec((_GATHER_W, c), index_map=lambda i: (i, 0))],
            core_axis_name=("c", "s"),
            dimension_semantics=(pltpu.PARALLEL,),
        )(i_hbm, o_hbm)

    return _gather(table, idx2)


def _decoder_body(h_ref, e_ref, g_ref, m_ref,
                  whT, wegT, w1T, w2T, b0, b1, b2,
                  d0T, db0, d1T, db1, n1w, n1b, n2w, n2b,
                  o_ref):
    k, b, c = e_ref.shape
    kb = k * b
    f32 = jnp.float32
    bf = jnp.bfloat16
    h = h_ref[...]
    e2 = e_ref[...].reshape(kb, c)
    g2 = g_ref[...].reshape(kb, c).astype(bf)
    eg = jnp.concatenate([e2, g2], axis=1)
    msk = m_ref[...]
    n_layers = whT.shape[0]
    for l in range(n_layers):
        hw = jnp.dot(h.astype(bf), whT[l], preferred_element_type=f32).astype(bf)
        epre = jnp.dot(eg, wegT[l], preferred_element_type=f32).astype(bf)
        x = epre.reshape(k, b, c) + hw[None] + b0[l][None]
        m1 = _gelu(x).reshape(kb, c)
        m2 = _gelu((jnp.dot(m1, w1T[l], preferred_element_type=f32)).astype(bf) + b1[l])
        m3 = jnp.dot(m2, w2T[l], preferred_element_type=f32).astype(bf)
        agg = (jnp.sum(m3.reshape(k, b, c), axis=0).astype(f32)
               + k * b2[l]) * (1.0 / _SCALE)
        h1 = _ln(h + agg, n1w[l], n1b[l])
        d = _gelu((jnp.dot(h1.astype(bf), d0T[l], preferred_element_type=f32)).astype(bf) + db0[l])
        d = jnp.dot(d, d1T[l], preferred_element_type=f32) + db1[l]
        h = _ln(h1 + d, n2w[l], n2b[l]) * msk
    o_ref[...] = h


def _run_decoder(h0, eds, gns, msk, w, base, chunk):
    n, c = h0.shape
    k = eds.shape[0]
    nb = chunk // _B

    def full3(a):
        return pl.BlockSpec(a.shape, lambda i: (0,) * a.ndim)

    in_specs = [
        pl.BlockSpec((_B, c), lambda i: (base + i, 0)),
        pl.BlockSpec((k, _B, eds.shape[2]), lambda i: (0, base + i, 0)),
        pl.BlockSpec((k, _B, gns.shape[2]), lambda i: (0, i, 0)),
        pl.BlockSpec((_B, 1), lambda i: (base + i, 0)),
    ] + [full3(a) for a in w]

    return pl.pallas_call(
        _decoder_body,
        grid=(nb,),
        in_specs=in_specs,
        out_specs=pl.BlockSpec((_B, c), lambda i: (i, 0)),
        out_shape=jax.ShapeDtypeStruct((chunk, c), h0.dtype),
        compiler_params=pltpu.CompilerParams(
            dimension_semantics=("parallel",),
        ),
    )(h0, eds, gns, msk, *w)


_N_CHUNKS = 5


def kernel(node_features, edge_features, neighbor_indices, mask, params):
    n, c = node_features.shape
    _, k, e = edge_features.shape

    eds = edge_features.transpose(1, 0, 2).astype(jnp.bfloat16)
    msk = mask.astype(node_features.dtype).reshape(n, 1)

    st = jnp.stack
    stb = lambda xs: jnp.stack(xs).astype(jnp.bfloat16)
    w = [
        stb([p['mW0'][:, :c].T for p in params]),
        stb([jnp.concatenate([p['mW0'][:, c:c + e],
                              p['mW0'][:, c + e + c:]], axis=1).T
             for p in params]),
        stb([p['mW1'].T for p in params]),
        stb([p['mW2'].T for p in params]),
        stb([p['mb0'][None] for p in params]),
        stb([p['mb1'][None] for p in params]),
        st([p['mb2'][None] for p in params]),
        stb([p['dW0'].T for p in params]),
        stb([p['db0'][None] for p in params]),
        stb([p['dW1'].T for p in params]),
        st([p['db1'][None] for p in params]),
        st([p['n1w'][None] for p in params]),
        st([p['n1b'][None] for p in params]),
        st([p['n2w'][None] for p in params]),
        st([p['n2b'][None] for p in params]),
    ]

    idx = neighbor_indices.T.reshape(-1).astype(jnp.int32)
    gns = _sc_gather(node_features, idx).reshape(k, n, c)
    return _run_decoder(node_features, eds, gns, msk, w, 0, n)
